# baseline (speedup 1.0000x reference)
import functools

import jax
import jax.numpy as jnp
from jax import lax
from jax.experimental import pallas as pl
from jax.experimental.pallas import tpu as pltpu
from jax.experimental.pallas import tpu_sc as plsc

_B = 16384
_D = 64
_CHUNK = 128
_G = 16
_UNROLL = 4


@functools.cache
def _make_sc_kernel():
  info = plsc.get_sparse_core_info()
  nc, ns = info.num_cores, info.num_subcores
  nw = nc * ns
  bpw = _B // nw
  nchunk = bpw // _CHUNK

  mesh = plsc.VectorSubcoreMesh(core_axis_name="c", subcore_axis_name="s")

  @functools.partial(
      pl.kernel,
      mesh=mesh,
      compiler_params=pltpu.CompilerParams(
          use_tc_tiling_on_sc=False, needs_layout_passes=False),
      out_type=jax.ShapeDtypeStruct((_B,), jnp.float32),
      scratch_types=[
          pltpu.VMEM((nchunk, _CHUNK), jnp.int32),
          pltpu.VMEM((nchunk, _CHUNK), jnp.int32),
          pltpu.VMEM((bpw, _D), jnp.float32),
          pltpu.VMEM((bpw, _D), jnp.float32),
          pltpu.VMEM((nchunk, _CHUNK), jnp.float32),
          pltpu.VMEM((nchunk, _CHUNK), jnp.float32),
          pltpu.VMEM((bpw,), jnp.float32),
          pltpu.SemaphoreType.DMA,
          pltpu.SemaphoreType.DMA,
          pltpu.SemaphoreType.DMA,
          pltpu.SemaphoreType.DMA,
      ],
  )
  def k(lhs_idx_hbm, rhs_idx_hbm, table_hbm, bias_lhs_hbm, bias_rhs_hbm,
        out_hbm, lhs_idx_v, rhs_idx_v, lhs_rows_v, rhs_rows_v, lb_v, rb_v,
        out_v, sem0, sem1, sem2, sem3):
    sems = [sem0, sem1, sem2, sem3]
    wid = lax.axis_index("s") * nc + lax.axis_index("c")
    pltpu.sync_copy(lhs_idx_hbm.at[wid], lhs_idx_v)
    pltpu.sync_copy(rhs_idx_hbm.at[wid], rhs_idx_v)

    copies = []
    for j in range(nchunk):
      copies.append([
          pltpu.async_copy(table_hbm.at[lhs_idx_v.at[j]],
                           lhs_rows_v.at[pl.ds(j * _CHUNK, _CHUNK)], sems[j]),
          pltpu.async_copy(table_hbm.at[rhs_idx_v.at[j]],
                           rhs_rows_v.at[pl.ds(j * _CHUNK, _CHUNK)], sems[j]),
          pltpu.async_copy(bias_lhs_hbm.at[lhs_idx_v.at[j]],
                           lb_v.at[j], sems[j]),
          pltpu.async_copy(bias_rhs_hbm.at[rhs_idx_v.at[j]],
                           rb_v.at[j], sems[j]),
      ])

    for j in range(nchunk):
      for c in copies[j]:
        c.wait()

      def group(g, carry, j=j):
        lane = lax.iota(jnp.int32, _G)
        sq_vec = jnp.zeros((_G,), jnp.float32)
        for u in range(_G):
          row = j * _CHUNK + g * _G + u
          acc = jnp.zeros((_G,), jnp.float32)
          for c in range(_D // _G):
            lv = lhs_rows_v[row, pl.ds(c * _G, _G)]
            rv = rhs_rows_v[row, pl.ds(c * _G, _G)]
            d = lv - rv
            acc = acc + d * d
          s = jnp.sum(acc)
          sq_vec = jnp.where(lane == u, jnp.full((_G,), s, jnp.float32),
                             sq_vec)
        lb = lb_v[j, pl.ds(g * _G, _G)]
        rb = rb_v[j, pl.ds(g * _G, _G)]
        out_v[pl.ds(j * _CHUNK + g * _G, _G)] = (lb + rb) - (sq_vec + 1e-12)
        return carry

      lax.fori_loop(0, _CHUNK // _G, group, 0)

    off = pl.multiple_of(wid * bpw, 8)
    pltpu.sync_copy(out_v, out_hbm.at[pl.ds(off, bpw)])

  return k, nw, nchunk


def kernel(input_triplet, table, bias_lhs, bias_rhs):
  k, nw, nchunk = _make_sc_kernel()
  lhs = input_triplet[:, 0].astype(jnp.int32).reshape(nw, nchunk, _CHUNK)
  rhs = input_triplet[:, -1].astype(jnp.int32).reshape(nw, nchunk, _CHUNK)
  return k(lhs, rhs, table, bias_lhs, bias_rhs)
